# baseline (speedup 1.0000x reference)
import jax
import jax.numpy as jnp
from jax import lax
from jax.experimental import pallas as pl
from jax.experimental.pallas import tpu as pltpu
from jax.experimental.pallas import tpu_sc as plsc

D = 16
NC = 2
NS = 16
NW = NC * NS
CHUNK = 5120


def _gather_body(table_hbm, idx_hbm, out_hbm, idx_v, rows_v, sem):
    b_per_w = idx_hbm.shape[0] // NW
    n_chunks = b_per_w // CHUNK
    wid = lax.axis_index("s") * NC + lax.axis_index("c")
    base = wid * b_per_w

    def step(i, carry):
        off = base + i * CHUNK
        pltpu.sync_copy(idx_hbm.at[pl.ds(off, CHUNK)], idx_v)
        pltpu.async_copy(table_hbm.at[idx_v], rows_v, sem).wait()
        pltpu.sync_copy(rows_v, out_hbm.at[pl.ds(off, CHUNK)])
        return carry

    lax.fori_loop(0, n_chunks, step, 0)


def kernel(embedding, mb_feats):
    batch, hist = mb_feats.shape
    n_idx = batch * hist
    idx = mb_feats.reshape(n_idx)

    mesh = plsc.VectorSubcoreMesh(core_axis_name="c", subcore_axis_name="s")
    gather = pl.kernel(
        _gather_body,
        out_type=jax.ShapeDtypeStruct((n_idx, D), jnp.float32),
        mesh=mesh,
        scratch_types=[
            pltpu.VMEM((CHUNK,), jnp.int32),
            pltpu.VMEM((CHUNK, D), jnp.float32),
            pltpu.SemaphoreType.DMA,
        ],
        compiler_params=pltpu.CompilerParams(use_tc_tiling_on_sc=False),
    )
    out = gather(embedding, idx)
    return lax.stop_gradient(out.reshape(batch, hist, D))


# speedup vs baseline: 1.0065x; 1.0065x over previous
import jax
import jax.numpy as jnp
from jax import lax
from jax.experimental import pallas as pl
from jax.experimental.pallas import tpu as pltpu
from jax.experimental.pallas import tpu_sc as plsc

D = 16
NC = 2
NS = 16
NW = NC * NS
CHUNK = 3200


def _gather_body(table_hbm, idx_hbm, out_hbm, idx_v, rows0, rows1, gsem0,
                 gsem1, osem0, osem1):
    n_chunks = idx_hbm.shape[1]
    b_per_w = n_chunks * CHUNK
    wid = lax.axis_index("s") * NC + lax.axis_index("c")
    base = wid * b_per_w

    rows = (rows0, rows1)
    gsem = (gsem0, gsem1)
    osem = (osem0, osem1)

    pltpu.sync_copy(idx_hbm.at[wid], idx_v)

    def start_gather(g, s):
        return pltpu.async_copy(table_hbm.at[idx_v.at[g]], rows[s], gsem[s])

    gd = [start_gather(0, 0), None]
    od = [None, None]
    for g in range(n_chunks):
        s = g & 1
        o = s ^ 1
        if g + 1 < n_chunks:
            if od[o] is not None:
                od[o].wait()
            gd[o] = start_gather(g + 1, o)
        gd[s].wait()
        od[s] = pltpu.async_copy(
            rows[s], out_hbm.at[pl.ds(base + g * CHUNK, CHUNK)], osem[s])
    od[(n_chunks - 1) & 1].wait()
    if n_chunks > 1:
        od[n_chunks & 1].wait()


def kernel(embedding, mb_feats):
    batch, hist = mb_feats.shape
    n_idx = batch * hist
    n_chunks = n_idx // (NW * CHUNK)
    idx = mb_feats.reshape(NW, n_chunks, CHUNK)

    mesh = plsc.VectorSubcoreMesh(core_axis_name="c", subcore_axis_name="s")
    gather = pl.kernel(
        _gather_body,
        out_type=jax.ShapeDtypeStruct((n_idx, D), jnp.float32),
        mesh=mesh,
        scratch_types=[
            pltpu.VMEM((n_chunks, CHUNK), jnp.int32),
            pltpu.VMEM((CHUNK, D), jnp.float32),
            pltpu.VMEM((CHUNK, D), jnp.float32),
            pltpu.SemaphoreType.DMA,
            pltpu.SemaphoreType.DMA,
            pltpu.SemaphoreType.DMA,
            pltpu.SemaphoreType.DMA,
        ],
        compiler_params=pltpu.CompilerParams(use_tc_tiling_on_sc=False),
    )
    out = gather(embedding, idx)
    return lax.stop_gradient(out.reshape(batch, hist, D))


# speedup vs baseline: 1.2495x; 1.2415x over previous
import jax
import jax.numpy as jnp
from jax import lax
from jax.experimental import pallas as pl
from jax.experimental.pallas import tpu as pltpu
from jax.experimental.pallas import tpu_sc as plsc

D = 16
NC = 2
NS = 16
NW = NC * NS
BB = 128


def _transpose_block(rows_v, trans_v):
    lane = lax.iota(jnp.int32, 16)
    for lb in range(8):
        row_ids = lane + (lb * 16)
        for c in range(D):
            col_ids = jnp.full((16,), c, jnp.int32)
            vals = plsc.load_gather(rows_v, [row_ids, col_ids])
            trans_v[c, pl.ds(lb * 16, 16)] = vals


def _gather_body(table_hbm, idx_hbm, out_hbm, idx_v, rows0, rows1, trans0,
                 trans1, gsem0, gsem1, osem0, osem1):
    hist = idx_hbm.shape[0]
    n_pairs = hist // 2
    bt = lax.axis_index("s") * NC + lax.axis_index("c")
    col = bt * BB

    pltpu.sync_copy(idx_hbm.at[:, pl.ds(col, BB)], idx_v.at[pl.ds(0, hist)])
    zeros = jnp.zeros((16,), jnp.int32)
    for j in range(BB // 16):
        idx_v[hist, pl.ds(j * 16, 16)] = zeros

    def gather(h, rows, sem):
        pltpu.async_copy(table_hbm.at[idx_v.at[h]], rows, sem)

    def wait_gather(rows, sem):
        pltpu.make_async_copy(table_hbm.at[idx_v.at[0]], rows, sem).wait()

    gather(0, rows0, gsem0)

    def step(i, carry):
        h0 = i * 2
        gather(h0 + 1, rows1, gsem1)
        wait_gather(rows0, gsem0)
        _transpose_block(rows0, trans0)
        o0 = pltpu.async_copy(trans0, out_hbm.at[h0, :, pl.ds(col, BB)], osem0)
        gather(h0 + 2, rows0, gsem0)
        wait_gather(rows1, gsem1)
        _transpose_block(rows1, trans1)
        o1 = pltpu.async_copy(trans1, out_hbm.at[h0 + 1, :, pl.ds(col, BB)],
                              osem1)
        o0.wait()
        o1.wait()
        return carry

    lax.fori_loop(0, n_pairs, step, 0)
    wait_gather(rows0, gsem0)


def kernel(embedding, mb_feats):
    batch, hist = mb_feats.shape
    idx_t = mb_feats.T

    mesh = plsc.VectorSubcoreMesh(core_axis_name="c", subcore_axis_name="s")
    gather = pl.kernel(
        _gather_body,
        out_type=jax.ShapeDtypeStruct((hist, D, batch), jnp.float32),
        mesh=mesh,
        scratch_types=[
            pltpu.VMEM((hist + 2, BB), jnp.int32),
            pltpu.VMEM((BB, D), jnp.float32),
            pltpu.VMEM((BB, D), jnp.float32),
            pltpu.VMEM((D, BB), jnp.float32),
            pltpu.VMEM((D, BB), jnp.float32),
            pltpu.SemaphoreType.DMA,
            pltpu.SemaphoreType.DMA,
            pltpu.SemaphoreType.DMA,
            pltpu.SemaphoreType.DMA,
        ],
        compiler_params=pltpu.CompilerParams(use_tc_tiling_on_sc=False,
                                             needs_layout_passes=False),
    )
    out_t = gather(embedding, idx_t)
    return lax.stop_gradient(out_t.transpose(2, 0, 1))


# speedup vs baseline: 1.2848x; 1.0282x over previous
import jax
import jax.numpy as jnp
from jax import lax
from jax.experimental import pallas as pl
from jax.experimental.pallas import tpu as pltpu
from jax.experimental.pallas import tpu_sc as plsc

D = 16
NC = 2
NS = 16
NW = NC * NS
BB = 128
HB = 10


def _transpose_slab(rows_v, trans_v):
    lane = lax.iota(jnp.int32, 16)
    for lb in range(8):
        row_ids = lane + (lb * 16)
        for c in range(D):
            col_ids = jnp.full((16,), c, jnp.int32)
            vals = plsc.load_gather(rows_v, [row_ids, col_ids])
            trans_v[c, pl.ds(lb * 16, 16)] = vals


def _gather_body(table_hbm, idx_hbm, out_hbm, idx_v, rows_a, rows_b, trans_a,
                 trans_b, gsem0, gsem1, osem0, osem1):
    hist = out_hbm.shape[0]
    n_blocks = hist // HB
    bt = lax.axis_index("s") * NC + lax.axis_index("c")
    col = bt * BB

    pltpu.sync_copy(idx_hbm.at[bt], idx_v)

    def gather(blk, rows, sem):
        safe = jnp.where(blk < n_blocks, blk, 0)
        pltpu.async_copy(
            table_hbm.at[idx_v.at[pl.ds(safe * (HB * BB), HB * BB)]], rows,
            sem)

    def wait_gather(rows, sem):
        pltpu.make_async_copy(table_hbm.at[idx_v.at[pl.ds(0, HB * BB)]], rows,
                              sem).wait()

    def wait_write(trans, sem, blk):
        pltpu.make_async_copy(
            trans, out_hbm.at[pl.ds(blk * HB, HB), :, pl.ds(col, BB)],
            sem).wait()

    def process(i, blk, rows, trans, gsem, osem):
        wait_gather(rows, gsem)

        def tpose(j2, carry):
            j = 2 * j2
            _transpose_slab(rows.at[pl.ds(j * BB, BB)], trans.at[j])
            _transpose_slab(rows.at[pl.ds((j + 1) * BB, BB)], trans.at[j + 1])
            return carry

        lax.fori_loop(0, HB // 2, tpose, 0)
        pltpu.async_copy(
            trans, out_hbm.at[pl.ds(blk * HB, HB), :, pl.ds(col, BB)], osem)

    gather(0, rows_a, gsem0)

    def step(i, carry):
        blk0 = i * 2
        gather(blk0 + 1, rows_b, gsem1)

        @pl.when(i > 0)
        def _():
            wait_write(trans_a, osem0, 0)

        process(i, blk0, rows_a, trans_a, gsem0, osem0)
        gather(blk0 + 2, rows_a, gsem0)

        @pl.when(i > 0)
        def _():
            wait_write(trans_b, osem1, 0)

        process(i, blk0 + 1, rows_b, trans_b, gsem1, osem1)
        return carry

    lax.fori_loop(0, n_blocks // 2, step, 0)
    wait_gather(rows_a, gsem0)
    wait_write(trans_a, osem0, 0)
    wait_write(trans_b, osem1, 0)


def kernel(embedding, mb_feats):
    batch, hist = mb_feats.shape
    idx_w = (mb_feats.T.reshape(hist, NW, BB).transpose(1, 0, 2)
             .reshape(NW, hist * BB))

    mesh = plsc.VectorSubcoreMesh(core_axis_name="c", subcore_axis_name="s")
    gather = pl.kernel(
        _gather_body,
        out_type=jax.ShapeDtypeStruct((hist, D, batch), jnp.float32),
        mesh=mesh,
        scratch_types=[
            pltpu.VMEM((hist * BB,), jnp.int32),
            pltpu.VMEM((HB * BB, D), jnp.float32),
            pltpu.VMEM((HB * BB, D), jnp.float32),
            pltpu.VMEM((HB, D, BB), jnp.float32),
            pltpu.VMEM((HB, D, BB), jnp.float32),
            pltpu.SemaphoreType.DMA,
            pltpu.SemaphoreType.DMA,
            pltpu.SemaphoreType.DMA,
            pltpu.SemaphoreType.DMA,
        ],
        compiler_params=pltpu.CompilerParams(use_tc_tiling_on_sc=False,
                                             needs_layout_passes=False),
    )
    out_t = gather(embedding, idx_w)
    return lax.stop_gradient(out_t.transpose(2, 0, 1))
